# baseline (speedup 1.0000x reference)
import jax
import jax.numpy as jnp
from jax.experimental import pallas as pl

_N_CFG, _N_PT, _N_FT = 10000, 100, 20


def _seg_mean(msg, dst, n):
    s = jax.ops.segment_sum(msg, dst, num_segments=n)
    cnt = jax.ops.segment_sum(jnp.ones((msg.shape[0],), msg.dtype), dst, num_segments=n)
    return s / jnp.maximum(cnt, 1.0)[:, None]


def _hetero_layer(h_cfg, h_pt, h_ft, W, edges):
    src_cc, dst_cc, src_cp, dst_cp, src_pc, dst_pc, src_cf, dst_cf, src_fc, dst_fc = edges
    m_cc = h_cfg @ W[0]
    m_cp = h_cfg @ W[1]
    m_pc = h_pt @ W[2]
    m_cf = h_cfg @ W[3]
    m_fc = h_ft @ W[4]
    agg_cfg = _seg_mean(jnp.take(m_cc, src_cc, axis=0), dst_cc, _N_CFG)
    agg_cfg = agg_cfg + _seg_mean(jnp.take(m_pc, src_pc, axis=0), dst_pc, _N_CFG)
    agg_cfg = agg_cfg + _seg_mean(jnp.take(m_fc, src_fc, axis=0), dst_fc, _N_CFG)
    agg_pt = _seg_mean(jnp.take(m_cp, src_cp, axis=0), dst_cp, _N_PT)
    agg_ft = _seg_mean(jnp.take(m_cf, src_cf, axis=0), dst_cf, _N_FT)
    return jax.nn.relu(agg_cfg), jax.nn.relu(agg_pt), jax.nn.relu(agg_ft)


def _dec_body(c_ref, w_ref, b_ref, logit_ref):
    logit_ref[...] = jnp.dot(c_ref[...], w_ref[...],
                             preferred_element_type=jnp.float32) + b_ref[...]


def kernel(label, content, src_cc, dst_cc, src_cp, dst_cp, src_pc, dst_pc, src_cf, dst_cf, src_fc, dst_fc, enc_label_W, enc_label_b, enc_content_W, enc_content_b, ptest_emb, ftest_emb, W1, W2, W3, W4, W5, dec_W, dec_b):
    edges = (src_cc, dst_cc, src_cp, dst_cp, src_pc, dst_pc, src_cf, dst_cf, src_fc, dst_fc)
    h_cfg = jnp.concatenate([label @ enc_label_W + enc_label_b,
                             content @ enc_content_W + enc_content_b], axis=-1)
    h_pt = jnp.tile(ptest_emb[None, :], (_N_PT, 1))
    h_ft = jnp.tile(ftest_emb[None, :], (_N_FT, 1))
    c1, p1, f1 = _hetero_layer(h_cfg, h_pt, h_ft, W1, edges)
    c2, p2, f2 = _hetero_layer(c1, p1, f1, W2, edges)
    c = jnp.concatenate([c1, c2], axis=-1)
    p = jnp.concatenate([p1, p2], axis=-1)
    f = jnp.concatenate([f1, f2], axis=-1)
    c3, p3, f3 = _hetero_layer(c, p, f, W3, edges)
    c4, p4, f4 = _hetero_layer(c3, p3, f3, W4, edges)
    c = jnp.concatenate([c3, c4], axis=-1)
    p = jnp.concatenate([p3, p4], axis=-1)
    f = jnp.concatenate([f3, f4], axis=-1)
    c5, p5, f5 = _hetero_layer(c, p, f, W5, edges)

    w_pad = jnp.zeros((128, 128), jnp.float32).at[:, :2].set(dec_W)
    b_pad = jnp.zeros((128,), jnp.float32).at[:2].set(dec_b)
    logits_pad = pl.pallas_call(
        _dec_body,
        out_shape=jax.ShapeDtypeStruct((_N_CFG, 128), jnp.float32),
    )(c5, w_pad, b_pad)
    logits = logits_pad[:, :2]
    pred = jax.nn.softmax(logits, axis=1)
    return logits, pred


# speedup vs baseline: 1.1497x; 1.1497x over previous
import jax
import jax.numpy as jnp
from jax import lax
from jax.experimental import pallas as pl
from jax.experimental.pallas import tpu as pltpu
from jax.experimental.pallas import tpu_sc as plsc

N_CFG, N_PT, N_FT = 10000, 100, 20
NPAD = 10240
NSM = 128
NTILES = 16
CHUNK = 128
SUP = 8
GRAN = NTILES * CHUNK * SUP
W = 128

EP_CC = 40 * GRAN
EP_CP = 8 * GRAN
EP_PC = 8 * GRAN
EP_CF = 2 * GRAN
EP_FC = 2 * GRAN

_BLK = 400
_GRID = N_CFG // _BLK
_ZROWS = NPAD // NTILES


def _pad_edges(src, dst, epad, sentinel):
    pe = epad - src.shape[0]
    s2 = jnp.concatenate([src, jnp.zeros((pe,), jnp.int32)]).reshape(-1, CHUNK)
    d2 = jnp.concatenate([dst, jnp.full((pe,), sentinel, jnp.int32)]).reshape(-1, CHUNK)
    return s2, d2


def _make_agg(col_split, with_counts):
    f32 = jnp.float32
    mesh = plsc.VectorSubcoreMesh(core_axis_name="c", subcore_axis_name="s",
                                  num_cores=2, num_subcores=NTILES)

    out_type = [
        jax.ShapeDtypeStruct((2, NPAD, W), f32),
        jax.ShapeDtypeStruct((2, NPAD, W), f32),
        jax.ShapeDtypeStruct((2, NPAD, W), f32),
        jax.ShapeDtypeStruct((2, NSM, W), f32),
        jax.ShapeDtypeStruct((2, NSM, W), f32),
    ]
    if with_counts:
        out_type += [
            jax.ShapeDtypeStruct((2, NPAD), f32),
            jax.ShapeDtypeStruct((2, NPAD), f32),
            jax.ShapeDtypeStruct((2, NPAD), f32),
            jax.ShapeDtypeStruct((2, NSM), f32),
            jax.ShapeDtypeStruct((2, NSM), f32),
        ]

    scratch = [
        pltpu.VMEM_SHARED((NPAD, W), f32),
        pltpu.VMEM_SHARED((NPAD,), f32),
        pltpu.VMEM((2, CHUNK, W), f32),
        pltpu.VMEM((SUP, CHUNK), jnp.int32),
        pltpu.VMEM((SUP, CHUNK), jnp.int32),
        pltpu.VMEM((CHUNK,), f32),
        pltpu.SemaphoreType.DMA,
        pltpu.SemaphoreType.DMA,
        pltpu.SemaphoreType.DMA,
        pltpu.SemaphoreType.DMA,
        pltpu.SemaphoreType.DMA,
    ]

    edge_div = GRAN if col_split else 2 * GRAN

    def body(*refs):
        (zrows, zcol,
         m_cc, m_pc, m_fc, m_cp, m_cf,
         scc2, dcc2, spc2, dpc2, sfc2, dfc2, scp2, dcp2, scf2, dcf2) = refs[:17]
        pos = 17
        s_cc_o, s_pc_o, s_fc_o, s_cp_o, s_cf_o = refs[pos:pos + 5]
        pos += 5
        if with_counts:
            c_cc_o, c_pc_o, c_fc_o, c_cp_o, c_cf_o = refs[pos:pos + 5]
            pos += 5
        (acc, cntacc, rbuf, sidx, didx, onesb,
         g0, g1, t0, t1, csem) = refs[pos:]
        gsem = (g0, g1)
        ssem = (t0, t1)

        c = lax.axis_index("c")
        s = lax.axis_index("s")

        one16 = jnp.ones((16,), f32)
        for k in range(CHUNK // 16):
            onesb[pl.ds(k * 16, 16)] = one16

        def zero_acc(nrows):
            per = nrows // NTILES
            r0 = s * per
            pltpu.sync_copy(zrows.at[pl.ds(0, per)], acc.at[pl.ds(r0, per)])
            if with_counts:
                pltpu.sync_copy(zcol.at[pl.ds(0, per)],
                                cntacc.at[pl.ds(r0, per)])

        def scatter_etype(src2, dst2, m_arr, nsup_tile, do_count):
            m_view = m_arr.at[c] if col_split else m_arr
            wid = s if col_split else c * NTILES + s
            base = wid * nsup_tile * SUP

            def sup(g, carry):
                row0 = base + g * SUP
                pltpu.sync_copy(src2.at[pl.ds(row0, SUP)], sidx)
                pltpu.sync_copy(dst2.at[pl.ds(row0, SUP)], didx)
                gat = [None, None]
                scat = [None, None]
                cds = []
                gat[0] = pltpu.async_copy(m_view.at[sidx.at[0]], rbuf.at[0],
                                          gsem[0])
                for j in range(SUP):
                    b = j % 2
                    nb = (j + 1) % 2
                    if j + 1 < SUP:
                        if j + 1 >= 2:
                            scat[nb].wait()
                        gat[nb] = pltpu.async_copy(m_view.at[sidx.at[j + 1]],
                                                   rbuf.at[nb], gsem[nb])
                    gat[b].wait()
                    scat[b] = pltpu.async_copy(rbuf.at[b], acc.at[didx.at[j]],
                                               ssem[b], add=True)
                    if do_count:
                        cds.append(pltpu.async_copy(onesb,
                                                    cntacc.at[didx.at[j]],
                                                    csem, add=True))
                scat[0].wait()
                scat[1].wait()
                for d in cds:
                    d.wait()
                return carry

            lax.fori_loop(0, nsup_tile, sup, 0)

        def drain(out_ref, nrows, cnt_o):
            per = nrows // NTILES
            r0 = s * per
            pltpu.sync_copy(acc.at[pl.ds(r0, per)],
                            out_ref.at[c].at[pl.ds(r0, per)])
            if with_counts and cnt_o is not None:
                pltpu.sync_copy(cntacc.at[pl.ds(r0, per)],
                                cnt_o.at[c].at[pl.ds(r0, per)])

        etypes = [
            (scc2, dcc2, m_cc, EP_CC // edge_div, NPAD, s_cc_o,
             c_cc_o if with_counts else None),
            (spc2, dpc2, m_pc, EP_PC // edge_div, NPAD, s_pc_o,
             c_pc_o if with_counts else None),
            (sfc2, dfc2, m_fc, EP_FC // edge_div, NPAD, s_fc_o,
             c_fc_o if with_counts else None),
            (scp2, dcp2, m_cp, EP_CP // edge_div, NSM, s_cp_o,
             c_cp_o if with_counts else None),
            (scf2, dcf2, m_cf, EP_CF // edge_div, NSM, s_cf_o,
             c_cf_o if with_counts else None),
        ]
        for src2, dst2, m_arr, nsup_tile, nrows, out_ref, cnt_o in etypes:
            zero_acc(nrows)
            plsc.subcore_barrier()
            scatter_etype(src2, dst2, m_arr, nsup_tile, cnt_o is not None)
            plsc.subcore_barrier()
            drain(out_ref, nrows, cnt_o)
            plsc.subcore_barrier()

    return pl.kernel(body, out_type=tuple(out_type), mesh=mesh,
                     scratch_types=tuple(scratch),
                     compiler_params=pltpu.CompilerParams(
                         use_tc_tiling_on_sc=False))


_agg_es_counts = _make_agg(False, True)
_agg_es = _make_agg(False, False)
_agg_cs = _make_agg(True, False)


def _enc_body(label_r, content_r, wl_r, bl_r, wc_r, bc_r, pe_r, fe_r, w1_r,
              mcc_r, mcp_r, mcf_r, mpc_r, mfc_r):
    hx = jnp.concatenate(
        [jnp.dot(label_r[...], wl_r[...], preferred_element_type=jnp.float32)
         + bl_r[...],
         jnp.dot(content_r[...], wc_r[...], preferred_element_type=jnp.float32)
         + bc_r[...]], axis=1)
    for k, out_r in ((0, mcc_r), (1, mcp_r), (3, mcf_r)):
        out_r[...] = jnp.dot(hx, w1_r[k], preferred_element_type=jnp.float32)
    yp = jnp.dot(pe_r[...], w1_r[2], preferred_element_type=jnp.float32)
    mpc_r[...] = jnp.broadcast_to(yp, (NSM, W))
    yf = jnp.dot(fe_r[...], w1_r[4], preferred_element_type=jnp.float32)
    mfc_r[...] = jnp.broadcast_to(yf, (NSM, W))


def _norm(s_val, c_val, cat):
    r = 1.0 / jnp.maximum(c_val[0] + c_val[1], 1.0)
    if cat:
        return jnp.concatenate([s_val[0] * r, s_val[1] * r], axis=1)
    return (s_val[0] + s_val[1]) * r


def _relu_x(scc, ccc, spc, cpc, sfc, cfc, cat):
    return jax.nn.relu(_norm(scc[...], ccc[...], cat)
                       + _norm(spc[...], cpc[...], cat)
                       + _norm(sfc[...], cfc[...], cat))


def _make_layer_body(cat_in, slab_out, has_prev, emit_x):
    def body(*refs):
        (scc, ccc, spc, cpc, sfc, cfc, scp, ccp, scf, ccf) = refs[:10]
        pos = 10
        if has_prev:
            cprev, pprev, fprev = refs[pos:pos + 3]
            pos += 3
        w_r = refs[pos]
        pos += 1
        mcc_r, mcp_r, mcf_r, mpc_r, mfc_r = refs[pos:pos + 5]
        pos += 5
        if emit_x:
            xc_o, xp_o, xf_o = refs[pos:pos + 3]

        def store(out_r, y):
            if slab_out:
                out_r[0] = y[:, :W]
                out_r[1] = y[:, W:]
            else:
                out_r[...] = y

        xn = _relu_x(scc, ccc, spc, cpc, sfc, cfc, cat_in)
        xc = jnp.concatenate([cprev[...], xn], axis=1) if has_prev else xn
        for k, out_r in ((0, mcc_r), (1, mcp_r), (3, mcf_r)):
            store(out_r, jnp.dot(xc, w_r[k],
                                 preferred_element_type=jnp.float32))

        xpn = jax.nn.relu(_norm(scp[...], ccp[...], cat_in))
        xp = jnp.concatenate([pprev[...], xpn], axis=1) if has_prev else xpn
        store(mpc_r, jnp.dot(xp, w_r[2], preferred_element_type=jnp.float32))

        xfn = jax.nn.relu(_norm(scf[...], ccf[...], cat_in))
        xf = jnp.concatenate([fprev[...], xfn], axis=1) if has_prev else xfn
        store(mfc_r, jnp.dot(xf, w_r[4], preferred_element_type=jnp.float32))

        if emit_x:
            xc_o[...] = xn
            xp_o[...] = xpn
            xf_o[...] = xfn
    return body


def _dec_body(scc, ccc, spc, cpc, sfc, cfc, w_r, b_r, logit_r, pred_r):
    c5 = _relu_x(scc, ccc, spc, cpc, sfc, cfc, False)
    lg = jnp.dot(c5, w_r[...], preferred_element_type=jnp.float32) + b_r[...]
    l0 = lg[:, 0:1]
    l1 = lg[:, 1:2]
    m = jnp.maximum(l0, l1)
    e0 = jnp.exp(l0 - m)
    e1 = jnp.exp(l1 - m)
    den = e0 + e1
    logit_r[...] = jnp.concatenate([l0, l1], axis=1)
    pred_r[...] = jnp.concatenate([e0 / den, e1 / den], axis=1)


def _cfg_spec():
    return pl.BlockSpec((2, _BLK, W), lambda i: (0, i, 0))


def _cnt_spec():
    return pl.BlockSpec((2, _BLK, 1), lambda i: (0, i, 0))


def _full(shape):
    nd = len(shape)
    return pl.BlockSpec(shape, lambda i, _n=nd: (0,) * _n)


def _m_specs(slab_out):
    if slab_out:
        cfg = pl.BlockSpec((2, _BLK, W), lambda i: (0, i, 0))
        sm = _full((2, NSM, W))
    else:
        cfg = pl.BlockSpec((_BLK, W), lambda i: (i, 0))
        sm = _full((NSM, W))
    return [cfg, cfg, cfg, sm, sm]


def _m_shapes(slab_out):
    f32 = jnp.float32
    if slab_out:
        return [jax.ShapeDtypeStruct((2, N_CFG, W), f32)] * 3 + \
               [jax.ShapeDtypeStruct((2, NSM, W), f32)] * 2
    return [jax.ShapeDtypeStruct((N_CFG, W), f32)] * 3 + \
           [jax.ShapeDtypeStruct((NSM, W), f32)] * 2


def _layer_call(svals, cvals, prevs, w, cat_in, slab_out, emit_x):
    f32 = jnp.float32
    in_specs = [_cfg_spec(), _cnt_spec(), _cfg_spec(), _cnt_spec(),
                _cfg_spec(), _cnt_spec(),
                _full((2, NSM, W)), _full((2, NSM, 1)),
                _full((2, NSM, W)), _full((2, NSM, 1))]
    args = [svals[0], cvals[0], svals[1], cvals[1], svals[2], cvals[2],
            svals[3], cvals[3], svals[4], cvals[4]]
    has_prev = prevs is not None
    if has_prev:
        ph = prevs[0].shape[1]
        in_specs += [pl.BlockSpec((_BLK, ph), lambda i: (i, 0)),
                     _full((NSM, ph)), _full((NSM, ph))]
        args += list(prevs)
    in_specs.append(_full(w.shape))
    args.append(w)

    out_specs = _m_specs(slab_out)
    out_shape = _m_shapes(slab_out)
    if emit_x:
        xw = W * (2 if cat_in else 1)
        out_specs += [pl.BlockSpec((_BLK, xw), lambda i: (i, 0)),
                      _full((NSM, xw)), _full((NSM, xw))]
        out_shape += [jax.ShapeDtypeStruct((N_CFG, xw), f32),
                      jax.ShapeDtypeStruct((NSM, xw), f32),
                      jax.ShapeDtypeStruct((NSM, xw), f32)]

    return pl.pallas_call(
        _make_layer_body(cat_in, slab_out, has_prev, emit_x),
        grid=(_GRID,),
        in_specs=in_specs,
        out_specs=out_specs,
        out_shape=out_shape,
    )(*args)


def kernel(label, content, src_cc, dst_cc, src_cp, dst_cp, src_pc, dst_pc,
           src_cf, dst_cf, src_fc, dst_fc, enc_label_W, enc_label_b,
           enc_content_W, enc_content_b, ptest_emb, ftest_emb,
           W1, W2, W3, W4, W5, dec_W, dec_b):
    f32 = jnp.float32
    scc2, dcc2 = _pad_edges(src_cc, dst_cc, EP_CC, N_CFG)
    scp2, dcp2 = _pad_edges(src_cp, dst_cp, EP_CP, N_PT)
    spc2, dpc2 = _pad_edges(src_pc, dst_pc, EP_PC, N_CFG)
    scf2, dcf2 = _pad_edges(src_cf, dst_cf, EP_CF, N_FT)
    sfc2, dfc2 = _pad_edges(src_fc, dst_fc, EP_FC, N_CFG)
    edges = (scc2, dcc2, spc2, dpc2, sfc2, dfc2, scp2, dcp2, scf2, dcf2)
    zrows = jnp.zeros((_ZROWS, W), f32)
    zcol = jnp.zeros((_ZROWS,), f32)

    m1 = pl.pallas_call(
        _enc_body,
        grid=(_GRID,),
        in_specs=[pl.BlockSpec((_BLK, 128), lambda i: (i, 0)),
                  pl.BlockSpec((_BLK, 128), lambda i: (i, 0)),
                  _full((128, 64)), _full((1, 64)),
                  _full((128, 64)), _full((1, 64)),
                  _full((1, 128)), _full((1, 128)),
                  _full((5, 128, 128))],
        out_specs=_m_specs(False),
        out_shape=_m_shapes(False),
    )(label, content, enc_label_W, enc_label_b.reshape(1, 64),
      enc_content_W, enc_content_b.reshape(1, 64),
      ptest_emb.reshape(1, 128), ftest_emb.reshape(1, 128), W1)

    r1 = _agg_es_counts(zrows, zcol, m1[0], m1[3], m1[4], m1[1], m1[2], *edges)
    s1 = r1[:5]
    cnts = tuple(x.reshape(2, -1, 1) for x in r1[5:])

    m2 = _layer_call(s1, cnts, None, W2, False, False, emit_x=True)
    s2 = _agg_es(zrows, zcol, m2[0], m2[3], m2[4], m2[1], m2[2], *edges)
    c1, p1, f1 = m2[5], m2[6], m2[7]

    m3 = _layer_call(s2, cnts, (c1, p1, f1), W3, False, True, emit_x=False)
    s3 = _agg_cs(zrows, zcol, m3[0], m3[3], m3[4], m3[1], m3[2], *edges)

    m4 = _layer_call(s3, cnts, None, W4, True, True, emit_x=True)
    s4 = _agg_cs(zrows, zcol, m4[0], m4[3], m4[4], m4[1], m4[2], *edges)
    c3, p3, f3 = m4[5], m4[6], m4[7]

    m5 = _layer_call(s4, cnts, (c3, p3, f3), W5, True, False, emit_x=False)
    s5 = _agg_es(zrows, zcol, m5[0], m5[3], m5[4], m5[1], m5[2], *edges)

    w_pad = jnp.zeros((128, 128), f32).at[:, :2].set(dec_W)
    b_pad = jnp.zeros((1, 128), f32).at[0, :2].set(dec_b)
    logits, pred = pl.pallas_call(
        _dec_body,
        grid=(_GRID,),
        in_specs=[_cfg_spec(), _cnt_spec(), _cfg_spec(), _cnt_spec(),
                  _cfg_spec(), _cnt_spec(),
                  _full((128, 128)), _full((1, 128))],
        out_specs=[pl.BlockSpec((_BLK, 2), lambda i: (i, 0)),
                   pl.BlockSpec((_BLK, 2), lambda i: (i, 0))],
        out_shape=[jax.ShapeDtypeStruct((N_CFG, 2), f32),
                   jax.ShapeDtypeStruct((N_CFG, 2), f32)],
    )(s5[0], cnts[0], s5[1], cnts[1], s5[2], cnts[2], w_pad, b_pad)
    return logits, pred


# speedup vs baseline: 4.6402x; 4.0359x over previous
import jax
import jax.numpy as jnp
from jax import lax
from jax.experimental import pallas as pl
from jax.experimental.pallas import tpu as pltpu
from jax.experimental.pallas import tpu_sc as plsc

N_CFG, N_PT, N_FT = 10000, 100, 20
NPAD = 10240
NSM = 128
NTILES = 16
NW = 32
CHUNK = 128
SUP = 8
GRAN = NTILES * CHUNK * SUP
W = 128

EP_CC = 40 * GRAN
EP_CP = 8 * GRAN
EP_PC = 8 * GRAN
EP_CF = 2 * GRAN
EP_FC = 2 * GRAN

_BLK = 400
_GRID = N_CFG // _BLK
_ZROWS = NPAD // NTILES

_PC_SZ = NPAD * 128
_CP_SZ = NSM * N_CFG
_ZFLAT = _PC_SZ // NW


def _pad_edges(src, dst, epad, sentinel):
    pe = epad - src.shape[0]
    s2 = jnp.concatenate([src, jnp.zeros((pe,), jnp.int32)]).reshape(-1, CHUNK)
    d2 = jnp.concatenate([dst, jnp.full((pe,), sentinel, jnp.int32)]).reshape(-1, CHUNK)
    return s2, d2


def _sc_mesh():
    return plsc.VectorSubcoreMesh(core_axis_name="c", subcore_axis_name="s",
                                  num_cores=2, num_subcores=NTILES)


def _make_agg(col_split, with_counts):
    f32 = jnp.float32
    out_type = [jax.ShapeDtypeStruct((2, NPAD, W), f32)]
    if with_counts:
        out_type.append(jax.ShapeDtypeStruct((2, NPAD), f32))

    scratch = [
        pltpu.VMEM_SHARED((NPAD, W), f32),
        pltpu.VMEM_SHARED((NPAD,), f32),
        pltpu.VMEM((2, CHUNK, W), f32),
        pltpu.VMEM((SUP, CHUNK), jnp.int32),
        pltpu.VMEM((SUP, CHUNK), jnp.int32),
        pltpu.VMEM((CHUNK,), f32),
        pltpu.SemaphoreType.DMA,
        pltpu.SemaphoreType.DMA,
        pltpu.SemaphoreType.DMA,
        pltpu.SemaphoreType.DMA,
        pltpu.SemaphoreType.DMA,
    ]

    nsup_tile = EP_CC // (GRAN if col_split else 2 * GRAN)

    def body(*refs):
        zrows, zcol, m_cc, scc2, dcc2 = refs[:5]
        pos = 5
        s_o = refs[pos]
        pos += 1
        if with_counts:
            cnt_o = refs[pos]
            pos += 1
        (acc, cntacc, rbuf, sidx, didx, onesb,
         g0, g1, t0, t1, csem) = refs[pos:]
        gsem = (g0, g1)
        ssem = (t0, t1)

        c = lax.axis_index("c")
        s = lax.axis_index("s")

        if with_counts:
            one16 = jnp.ones((16,), f32)
            for k in range(CHUNK // 16):
                onesb[pl.ds(k * 16, 16)] = one16

        r0 = s * _ZROWS
        pltpu.sync_copy(zrows.at[pl.ds(0, _ZROWS)], acc.at[pl.ds(r0, _ZROWS)])
        if with_counts:
            pltpu.sync_copy(zcol.at[pl.ds(0, _ZROWS)],
                            cntacc.at[pl.ds(r0, _ZROWS)])
        plsc.subcore_barrier()

        m_view = m_cc.at[c] if col_split else m_cc
        wid = s if col_split else c * NTILES + s
        base = wid * nsup_tile * SUP

        def sup(g, carry):
            row0 = base + g * SUP
            pltpu.sync_copy(scc2.at[pl.ds(row0, SUP)], sidx)
            pltpu.sync_copy(dcc2.at[pl.ds(row0, SUP)], didx)
            gat = [None, None]
            scat = [None, None]
            cds = []
            gat[0] = pltpu.async_copy(m_view.at[sidx.at[0]], rbuf.at[0],
                                      gsem[0])
            for j in range(SUP):
                b = j % 2
                nb = (j + 1) % 2
                if j + 1 < SUP:
                    if j + 1 >= 2:
                        scat[nb].wait()
                    gat[nb] = pltpu.async_copy(m_view.at[sidx.at[j + 1]],
                                               rbuf.at[nb], gsem[nb])
                gat[b].wait()
                scat[b] = pltpu.async_copy(rbuf.at[b], acc.at[didx.at[j]],
                                           ssem[b], add=True)
                if with_counts:
                    cds.append(pltpu.async_copy(onesb, cntacc.at[didx.at[j]],
                                                csem, add=True))
            scat[0].wait()
            scat[1].wait()
            for d in cds:
                d.wait()
            return carry

        lax.fori_loop(0, nsup_tile, sup, 0)
        plsc.subcore_barrier()

        pltpu.sync_copy(acc.at[pl.ds(r0, _ZROWS)],
                        s_o.at[c].at[pl.ds(r0, _ZROWS)])
        if with_counts:
            pltpu.sync_copy(cntacc.at[pl.ds(r0, _ZROWS)],
                            cnt_o.at[c].at[pl.ds(r0, _ZROWS)])

    ot = tuple(out_type) if len(out_type) > 1 else out_type[0]
    return pl.kernel(body, out_type=ot, mesh=_sc_mesh(),
                     scratch_types=tuple(scratch),
                     compiler_params=pltpu.CompilerParams(
                         use_tc_tiling_on_sc=False))


_agg_es_counts = _make_agg(False, True)
_agg_es = _make_agg(False, False)
_agg_cs = _make_agg(True, False)


def _cmat_kernel():
    f32 = jnp.float32
    out_type = [
        jax.ShapeDtypeStruct((2, _PC_SZ), f32),
        jax.ShapeDtypeStruct((2, _PC_SZ), f32),
        jax.ShapeDtypeStruct((2, _CP_SZ), f32),
        jax.ShapeDtypeStruct((2, _CP_SZ), f32),
    ]
    scratch = [
        pltpu.VMEM_SHARED((_PC_SZ,), f32),
        pltpu.VMEM((SUP, CHUNK), jnp.int32),
        pltpu.VMEM((SUP, CHUNK), jnp.int32),
        pltpu.VMEM((SUP, CHUNK), jnp.int32),
        pltpu.VMEM((CHUNK,), f32),
        pltpu.SemaphoreType.DMA,
    ]

    def body(zflat, spc2, dpc2, sfc2, dfc2, scp2, dcp2, scf2, dcf2,
             pc_o, fc_o, cp_o, cf_o, acc, sidx, didx, fbuf, onesb, csem):
        c = lax.axis_index("c")
        s = lax.axis_index("s")
        wid = c * NTILES + s

        one16 = jnp.ones((16,), f32)
        for k in range(CHUNK // 16):
            onesb[pl.ds(k * 16, 16)] = one16

        def phase(src2, dst2, nsup_tile, ncols, n, out_ref):
            ntile = n // NW
            pltpu.sync_copy(zflat.at[pl.ds(0, ntile)],
                            acc.at[pl.ds(s * (n // NTILES), ntile)])
            pltpu.sync_copy(zflat.at[pl.ds(0, ntile)],
                            acc.at[pl.ds(s * (n // NTILES) + ntile, ntile)])
            plsc.subcore_barrier()

            base = (c * NTILES + s) * nsup_tile * SUP

            def sup(g, carry):
                row0 = base + g * SUP
                pltpu.sync_copy(src2.at[pl.ds(row0, SUP)], sidx)
                pltpu.sync_copy(dst2.at[pl.ds(row0, SUP)], didx)
                for j in range(SUP):
                    for k in range(CHUNK // 16):
                        s16 = sidx[j, pl.ds(k * 16, 16)]
                        d16 = didx[j, pl.ds(k * 16, 16)]
                        fbuf[j, pl.ds(k * 16, 16)] = d16 * ncols + s16
                descs = [pltpu.async_copy(onesb, acc.at[fbuf.at[j]],
                                          csem, add=True)
                         for j in range(SUP)]
                for d in descs:
                    d.wait()
                return carry

            lax.fori_loop(0, nsup_tile, sup, 0)
            plsc.subcore_barrier()
            pltpu.sync_copy(acc.at[pl.ds(s * (n // NTILES), n // NTILES)],
                            out_ref.at[c].at[pl.ds(s * (n // NTILES),
                                                   n // NTILES)])
            plsc.subcore_barrier()

        phase(spc2, dpc2, EP_PC // (2 * GRAN), 128, _PC_SZ, pc_o)
        phase(sfc2, dfc2, EP_FC // (2 * GRAN), 128, _PC_SZ, fc_o)
        phase(scp2, dcp2, EP_CP // (2 * GRAN), N_CFG, _CP_SZ, cp_o)
        phase(scf2, dcf2, EP_CF // (2 * GRAN), N_CFG, _CP_SZ, cf_o)

    return pl.kernel(body, out_type=tuple(out_type), mesh=_sc_mesh(),
                     scratch_types=tuple(scratch),
                     compiler_params=pltpu.CompilerParams(
                         use_tc_tiling_on_sc=False))


_cmat = _cmat_kernel()


def _enc_body(label_r, content_r, wl_r, bl_r, wc_r, bc_r, pe_r, fe_r, w1_r,
              mcc_r, mcp_r, mcf_r, mpc_r, mfc_r):
    hx = jnp.concatenate(
        [jnp.dot(label_r[...], wl_r[...], preferred_element_type=jnp.float32)
         + bl_r[...],
         jnp.dot(content_r[...], wc_r[...], preferred_element_type=jnp.float32)
         + bc_r[...]], axis=1)
    for k, out_r in ((0, mcc_r), (1, mcp_r), (3, mcf_r)):
        out_r[...] = jnp.dot(hx, w1_r[k], preferred_element_type=jnp.float32)
    yp = jnp.dot(pe_r[...], w1_r[2], preferred_element_type=jnp.float32)
    mpc_r[...] = jnp.broadcast_to(yp, (NSM, W))
    yf = jnp.dot(fe_r[...], w1_r[4], preferred_element_type=jnp.float32)
    mfc_r[...] = jnp.broadcast_to(yf, (NSM, W))


def _norm_cc(s_val, c_val, cat):
    r = 1.0 / jnp.maximum(c_val[0] + c_val[1], 1.0)
    if cat:
        return jnp.concatenate([s_val[0] * r, s_val[1] * r], axis=1)
    return (s_val[0] + s_val[1]) * r


def _cmm(c_ref, m_ref):
    cv = c_ref[...]
    cm = cv[0] + cv[1]
    r = 1.0 / jnp.maximum(jnp.sum(cm, axis=1, keepdims=True), 1.0)
    return jnp.dot(cm, m_ref[...], preferred_element_type=jnp.float32) * r


def _make_big_body(cat_in, slab_out, has_prev, emit_x, last):
    def body(*refs):
        scc, ccc, cpc_r, cfc_r, mpc_r, mfc_r = refs[:6]
        pos = 6
        if has_prev:
            cprev = refs[pos]
            pos += 1
        w_r = refs[pos]
        pos += 1
        outs = refs[pos:]

        xn = jax.nn.relu(_norm_cc(scc[...], ccc[...], cat_in)
                         + _cmm(cpc_r, mpc_r) + _cmm(cfc_r, mfc_r))
        xc = jnp.concatenate([cprev[...], xn], axis=1) if has_prev else xn

        def store(out_r, y):
            if slab_out:
                out_r[0] = y[:, :W]
                out_r[1] = y[:, W:]
            else:
                out_r[...] = y

        store(outs[0], jnp.dot(xc, w_r[0], preferred_element_type=jnp.float32))
        if not last:
            outs[1][...] = jnp.dot(xc, w_r[1],
                                   preferred_element_type=jnp.float32)
            outs[2][...] = jnp.dot(xc, w_r[3],
                                   preferred_element_type=jnp.float32)
        if emit_x:
            outs[3][...] = xn
    return body


def _make_small_body(has_prev, emit_x):
    def body(*refs):
        ccp_r, ccf_r, mcp_r, mcf_r = refs[:4]
        pos = 4
        if has_prev:
            pprev, fprev = refs[pos:pos + 2]
            pos += 2
        w_r = refs[pos]
        pos += 1
        outs = refs[pos:]

        xpn = jax.nn.relu(_cmm(ccp_r, mcp_r))
        xfn = jax.nn.relu(_cmm(ccf_r, mcf_r))
        xp = jnp.concatenate([pprev[...], xpn], axis=1) if has_prev else xpn
        xf = jnp.concatenate([fprev[...], xfn], axis=1) if has_prev else xfn
        outs[0][...] = jnp.dot(xp, w_r[2], preferred_element_type=jnp.float32)
        outs[1][...] = jnp.dot(xf, w_r[4], preferred_element_type=jnp.float32)
        if emit_x:
            outs[2][...] = xpn
            outs[3][...] = xfn
    return body


def _dec_body(scc, ccc, cpc_r, cfc_r, mpc_r, mfc_r, w_r, b_r,
              logit_r, pred_r):
    c5 = jax.nn.relu(_norm_cc(scc[...], ccc[...], False)
                     + _cmm(cpc_r, mpc_r) + _cmm(cfc_r, mfc_r))
    lg = jnp.dot(c5, w_r[...], preferred_element_type=jnp.float32) + b_r[...]
    l0 = lg[:, 0:1]
    l1 = lg[:, 1:2]
    m = jnp.maximum(l0, l1)
    e0 = jnp.exp(l0 - m)
    e1 = jnp.exp(l1 - m)
    den = e0 + e1
    logit_r[...] = jnp.concatenate([l0, l1], axis=1)
    pred_r[...] = jnp.concatenate([e0 / den, e1 / den], axis=1)


def _cfg_spec():
    return pl.BlockSpec((2, _BLK, W), lambda i: (0, i, 0))


def _cnt_spec():
    return pl.BlockSpec((2, _BLK, 1), lambda i: (0, i, 0))


def _row_spec(width):
    return pl.BlockSpec((_BLK, width), lambda i: (i, 0))


def _full(shape):
    nd = len(shape)
    return pl.BlockSpec(shape, lambda i, _n=nd: (0,) * _n)


def _big_call(s_cc, cnt3, c_pc, c_fc, m_pc, m_fc, cprev, w,
              cat_in, slab_out, emit_x, last, win, wout):
    f32 = jnp.float32
    in_specs = [_cfg_spec(), _cnt_spec(), _cfg_spec(), _cfg_spec(),
                _full((NSM, win)), _full((NSM, win))]
    args = [s_cc, cnt3, c_pc, c_fc, m_pc, m_fc]
    has_prev = cprev is not None
    if has_prev:
        in_specs.append(_row_spec(cprev.shape[1]))
        args.append(cprev)
    in_specs.append(_full(w.shape))
    args.append(w)

    if slab_out:
        mcc_spec = pl.BlockSpec((2, _BLK, W), lambda i: (0, i, 0))
        mcc_shape = jax.ShapeDtypeStruct((2, N_CFG, W), f32)
    else:
        mcc_spec = _row_spec(wout)
        mcc_shape = jax.ShapeDtypeStruct((N_CFG, wout), f32)
    out_specs = [mcc_spec]
    out_shape = [mcc_shape]
    if not last:
        out_specs += [_row_spec(wout), _row_spec(wout)]
        out_shape += [jax.ShapeDtypeStruct((N_CFG, wout), f32)] * 2
    if emit_x:
        xw = W * (2 if cat_in else 1)
        out_specs.append(_row_spec(xw))
        out_shape.append(jax.ShapeDtypeStruct((N_CFG, xw), f32))

    return pl.pallas_call(
        _make_big_body(cat_in, slab_out, has_prev, emit_x, last),
        grid=(_GRID,),
        in_specs=in_specs,
        out_specs=out_specs,
        out_shape=out_shape,
    )(*args)


def _small_call(c_cp, c_cf, m_cp, m_cf, prevs, w, emit_x, wout, xw):
    f32 = jnp.float32
    args = [c_cp, c_cf, m_cp, m_cf]
    has_prev = prevs is not None
    if has_prev:
        args += list(prevs)
    args.append(w)
    out_shape = [jax.ShapeDtypeStruct((NSM, wout), f32)] * 2
    if emit_x:
        out_shape += [jax.ShapeDtypeStruct((NSM, xw), f32)] * 2
    return pl.pallas_call(
        _make_small_body(has_prev, emit_x),
        out_shape=out_shape,
    )(*args)


def kernel(label, content, src_cc, dst_cc, src_cp, dst_cp, src_pc, dst_pc,
           src_cf, dst_cf, src_fc, dst_fc, enc_label_W, enc_label_b,
           enc_content_W, enc_content_b, ptest_emb, ftest_emb,
           W1, W2, W3, W4, W5, dec_W, dec_b):
    f32 = jnp.float32
    scc2, dcc2 = _pad_edges(src_cc, dst_cc, EP_CC, N_CFG)
    scp2, dcp2 = _pad_edges(src_cp, dst_cp, EP_CP, N_PT)
    spc2, dpc2 = _pad_edges(src_pc, dst_pc, EP_PC, N_CFG)
    scf2, dcf2 = _pad_edges(src_cf, dst_cf, EP_CF, N_FT)
    sfc2, dfc2 = _pad_edges(src_fc, dst_fc, EP_FC, N_CFG)
    zrows = jnp.zeros((_ZROWS, W), f32)
    zcol = jnp.zeros((_ZROWS,), f32)
    zflat = jnp.zeros((_ZFLAT,), f32)

    pc_f, fc_f, cp_f, cf_f = _cmat(zflat, spc2, dpc2, sfc2, dfc2,
                                   scp2, dcp2, scf2, dcf2)
    c_pc = pc_f.reshape(2, NPAD, 128)
    c_fc = fc_f.reshape(2, NPAD, 128)
    c_cp = cp_f.reshape(2, NSM, N_CFG)
    c_cf = cf_f.reshape(2, NSM, N_CFG)

    m1 = pl.pallas_call(
        _enc_body,
        grid=(_GRID,),
        in_specs=[pl.BlockSpec((_BLK, 128), lambda i: (i, 0)),
                  pl.BlockSpec((_BLK, 128), lambda i: (i, 0)),
                  _full((128, 64)), _full((1, 64)),
                  _full((128, 64)), _full((1, 64)),
                  _full((1, 128)), _full((1, 128)),
                  _full((5, 128, 128))],
        out_specs=[_row_spec(128), _row_spec(128), _row_spec(128),
                   _full((NSM, 128)), _full((NSM, 128))],
        out_shape=[jax.ShapeDtypeStruct((N_CFG, 128), f32)] * 3
                  + [jax.ShapeDtypeStruct((NSM, 128), f32)] * 2,
    )(label, content, enc_label_W, enc_label_b.reshape(1, 64),
      enc_content_W, enc_content_b.reshape(1, 64),
      ptest_emb.reshape(1, 128), ftest_emb.reshape(1, 128), W1)
    m1_cc, m1_cp, m1_cf, m1_pc, m1_fc = m1

    s1, cnt_cc = _agg_es_counts(zrows, zcol, m1_cc, scc2, dcc2)
    cnt3 = cnt_cc.reshape(2, NPAD, 1)

    m2 = _big_call(s1, cnt3, c_pc, c_fc, m1_pc, m1_fc, None, W2,
                   False, False, True, False, 128, 128)
    m2_cc, m2_cp, m2_cf, c1 = m2
    sm2 = _small_call(c_cp, c_cf, m1_cp, m1_cf, None, W2, True, 128, 128)
    m2_pc, m2_fc, p1, f1 = sm2
    s2 = _agg_es(zrows, zcol, m2_cc, scc2, dcc2)

    m3 = _big_call(s2, cnt3, c_pc, c_fc, m2_pc, m2_fc, c1, W3,
                   False, True, False, False, 128, 256)
    m3_cc, m3_cp, m3_cf = m3
    sm3 = _small_call(c_cp, c_cf, m2_cp, m2_cf, (p1, f1), W3, False, 256, 0)
    m3_pc, m3_fc = sm3
    s3 = _agg_cs(zrows, zcol, m3_cc, scc2, dcc2)

    m4 = _big_call(s3, cnt3, c_pc, c_fc, m3_pc, m3_fc, None, W4,
                   True, True, True, False, 256, 256)
    m4_cc, m4_cp, m4_cf, c3 = m4
    sm4 = _small_call(c_cp, c_cf, m3_cp, m3_cf, None, W4, True, 256, 256)
    m4_pc, m4_fc, p3, f3 = sm4
    s4 = _agg_cs(zrows, zcol, m4_cc, scc2, dcc2)

    m5 = _big_call(s4, cnt3, c_pc, c_fc, m4_pc, m4_fc, c3, W5,
                   True, False, False, True, 256, 128)
    m5_cc = m5[0]
    sm5 = _small_call(c_cp, c_cf, m4_cp, m4_cf, (p3, f3), W5, False, 128, 0)
    m5_pc, m5_fc = sm5
    s5 = _agg_es(zrows, zcol, m5_cc, scc2, dcc2)

    w_pad = jnp.zeros((128, 128), f32).at[:, :2].set(dec_W)
    b_pad = jnp.zeros((1, 128), f32).at[0, :2].set(dec_b)
    logits, pred = pl.pallas_call(
        _dec_body,
        grid=(_GRID,),
        in_specs=[_cfg_spec(), _cnt_spec(), _cfg_spec(), _cfg_spec(),
                  _full((NSM, 128)), _full((NSM, 128)),
                  _full((128, 128)), _full((1, 128))],
        out_specs=[pl.BlockSpec((_BLK, 2), lambda i: (i, 0)),
                   pl.BlockSpec((_BLK, 2), lambda i: (i, 0))],
        out_shape=[jax.ShapeDtypeStruct((N_CFG, 2), f32),
                   jax.ShapeDtypeStruct((N_CFG, 2), f32)],
    )(s5, cnt3, c_pc, c_fc, m5_pc, m5_fc, w_pad, b_pad)
    return logits, pred


# speedup vs baseline: 12.9118x; 2.7826x over previous
import jax
import jax.numpy as jnp
from jax import lax
from jax.experimental import pallas as pl
from jax.experimental.pallas import tpu as pltpu
from jax.experimental.pallas import tpu_sc as plsc

N_CFG, N_PT, N_FT = 10000, 100, 20
NPAD = 10240
NSM = 128
NTILES = 16
NW = 32
CHUNK = 128
SUP = 8
GRAN = NTILES * CHUNK * SUP
W = 128

EP_CC = 40 * GRAN
EP_CP = 8 * GRAN
EP_PC = 8 * GRAN
EP_CF = 2 * GRAN
EP_FC = 2 * GRAN

_BLK = 400
_GRID = N_CFG // _BLK
_ZROWS = NPAD // NTILES

_PC_SZ = NPAD * 128
_CP_SZ = NSM * N_CFG
_ZFLAT = _PC_SZ // NW


def _pad_edges(src, dst, epad, sent_base, nsent, src_mod):
    pe = epad - src.shape[0]
    ar = jnp.arange(pe, dtype=jnp.int32)
    s2 = jnp.concatenate([src, ar % src_mod]).reshape(-1, CHUNK)
    d2 = jnp.concatenate([dst, sent_base + ar % nsent]).reshape(-1, CHUNK)
    return s2, d2


def _sc_mesh():
    return plsc.VectorSubcoreMesh(core_axis_name="c", subcore_axis_name="s",
                                  num_cores=2, num_subcores=NTILES)


def _make_agg(col_split, with_counts):
    f32 = jnp.float32
    out_type = [jax.ShapeDtypeStruct((2, NPAD, W), f32)]
    if with_counts:
        out_type.append(jax.ShapeDtypeStruct((2, NPAD), f32))

    scratch = [
        pltpu.VMEM_SHARED((NPAD, W), f32),
        pltpu.VMEM_SHARED((NPAD,), f32),
        pltpu.VMEM((2, CHUNK, W), f32),
        pltpu.VMEM((SUP, CHUNK), jnp.int32),
        pltpu.VMEM((SUP, CHUNK), jnp.int32),
        pltpu.VMEM((CHUNK,), f32),
        pltpu.SemaphoreType.DMA,
        pltpu.SemaphoreType.DMA,
        pltpu.SemaphoreType.DMA,
        pltpu.SemaphoreType.DMA,
        pltpu.SemaphoreType.DMA,
    ]

    nsup_tile = EP_CC // (GRAN if col_split else 2 * GRAN)

    def body(*refs):
        zrows, zcol, m_cc, scc2, dcc2 = refs[:5]
        pos = 5
        s_o = refs[pos]
        pos += 1
        if with_counts:
            cnt_o = refs[pos]
            pos += 1
        (acc, cntacc, rbuf, sidx, didx, onesb,
         g0, g1, t0, t1, csem) = refs[pos:]
        gsem = (g0, g1)
        ssem = (t0, t1)

        c = lax.axis_index("c")
        s = lax.axis_index("s")

        if with_counts:
            one16 = jnp.ones((16,), f32)
            for k in range(CHUNK // 16):
                onesb[pl.ds(k * 16, 16)] = one16

        r0 = s * _ZROWS
        pltpu.sync_copy(zrows.at[pl.ds(0, _ZROWS)], acc.at[pl.ds(r0, _ZROWS)])
        if with_counts:
            pltpu.sync_copy(zcol.at[pl.ds(0, _ZROWS)],
                            cntacc.at[pl.ds(r0, _ZROWS)])
        plsc.subcore_barrier()

        m_view = m_cc.at[c] if col_split else m_cc
        wid = s if col_split else c * NTILES + s
        base = wid * nsup_tile * SUP

        def sup(g, carry):
            row0 = base + g * SUP
            pltpu.sync_copy(scc2.at[pl.ds(row0, SUP)], sidx)
            pltpu.sync_copy(dcc2.at[pl.ds(row0, SUP)], didx)
            gat = [None, None]
            scat = [None, None]
            cds = []
            gat[0] = pltpu.async_copy(m_view.at[sidx.at[0]], rbuf.at[0],
                                      gsem[0])
            for j in range(SUP):
                b = j % 2
                nb = (j + 1) % 2
                if j + 1 < SUP:
                    if j + 1 >= 2:
                        scat[nb].wait()
                    gat[nb] = pltpu.async_copy(m_view.at[sidx.at[j + 1]],
                                               rbuf.at[nb], gsem[nb])
                gat[b].wait()
                scat[b] = pltpu.async_copy(rbuf.at[b], acc.at[didx.at[j]],
                                           ssem[b], add=True)
                if with_counts:
                    cds.append(pltpu.async_copy(onesb, cntacc.at[didx.at[j]],
                                                csem, add=True))
            scat[0].wait()
            scat[1].wait()
            for d in cds:
                d.wait()
            return carry

        lax.fori_loop(0, nsup_tile, sup, 0)
        plsc.subcore_barrier()

        pltpu.sync_copy(acc.at[pl.ds(r0, _ZROWS)],
                        s_o.at[c].at[pl.ds(r0, _ZROWS)])
        if with_counts:
            pltpu.sync_copy(cntacc.at[pl.ds(r0, _ZROWS)],
                            cnt_o.at[c].at[pl.ds(r0, _ZROWS)])

    ot = tuple(out_type) if len(out_type) > 1 else out_type[0]
    return pl.kernel(body, out_type=ot, mesh=_sc_mesh(),
                     scratch_types=tuple(scratch),
                     compiler_params=pltpu.CompilerParams(
                         use_tc_tiling_on_sc=False))


_agg_es_counts = _make_agg(False, True)
_agg_es = _make_agg(False, False)
_agg_cs = _make_agg(True, False)


def _cmat_kernel():
    f32 = jnp.float32
    out_type = [
        jax.ShapeDtypeStruct((2, _PC_SZ), f32),
        jax.ShapeDtypeStruct((2, _PC_SZ), f32),
        jax.ShapeDtypeStruct((2, _CP_SZ), f32),
        jax.ShapeDtypeStruct((2, _CP_SZ), f32),
    ]
    scratch = [
        pltpu.VMEM_SHARED((_PC_SZ,), f32),
        pltpu.VMEM((SUP, CHUNK), jnp.int32),
        pltpu.VMEM((SUP, CHUNK), jnp.int32),
        pltpu.VMEM((SUP, CHUNK), jnp.int32),
        pltpu.VMEM((CHUNK,), f32),
        pltpu.SemaphoreType.DMA,
    ]

    def body(zflat, spc2, dpc2, sfc2, dfc2, scp2, dcp2, scf2, dcf2,
             pc_o, fc_o, cp_o, cf_o, acc, sidx, didx, fbuf, onesb, csem):
        c = lax.axis_index("c")
        s = lax.axis_index("s")
        wid = c * NTILES + s

        one16 = jnp.ones((16,), f32)
        for k in range(CHUNK // 16):
            onesb[pl.ds(k * 16, 16)] = one16

        def phase(src2, dst2, nsup_tile, ncols, n, out_ref):
            ntile = n // NW
            pltpu.sync_copy(zflat.at[pl.ds(0, ntile)],
                            acc.at[pl.ds(s * (n // NTILES), ntile)])
            pltpu.sync_copy(zflat.at[pl.ds(0, ntile)],
                            acc.at[pl.ds(s * (n // NTILES) + ntile, ntile)])
            plsc.subcore_barrier()

            base = (c * NTILES + s) * nsup_tile * SUP

            def sup(g, carry):
                row0 = base + g * SUP
                pltpu.sync_copy(src2.at[pl.ds(row0, SUP)], sidx)
                pltpu.sync_copy(dst2.at[pl.ds(row0, SUP)], didx)
                for j in range(SUP):
                    for k in range(CHUNK // 16):
                        s16 = sidx[j, pl.ds(k * 16, 16)]
                        d16 = didx[j, pl.ds(k * 16, 16)]
                        fbuf[j, pl.ds(k * 16, 16)] = d16 * ncols + s16
                descs = [pltpu.async_copy(onesb, acc.at[fbuf.at[j]],
                                          csem, add=True)
                         for j in range(SUP)]
                for d in descs:
                    d.wait()
                return carry

            lax.fori_loop(0, nsup_tile, sup, 0)
            plsc.subcore_barrier()
            pltpu.sync_copy(acc.at[pl.ds(s * (n // NTILES), n // NTILES)],
                            out_ref.at[c].at[pl.ds(s * (n // NTILES),
                                                   n // NTILES)])
            plsc.subcore_barrier()

        phase(spc2, dpc2, EP_PC // (2 * GRAN), 128, _PC_SZ, pc_o)
        phase(sfc2, dfc2, EP_FC // (2 * GRAN), 128, _PC_SZ, fc_o)
        phase(scp2, dcp2, EP_CP // (2 * GRAN), N_CFG, _CP_SZ, cp_o)
        phase(scf2, dcf2, EP_CF // (2 * GRAN), N_CFG, _CP_SZ, cf_o)

    return pl.kernel(body, out_type=tuple(out_type), mesh=_sc_mesh(),
                     scratch_types=tuple(scratch),
                     compiler_params=pltpu.CompilerParams(
                         use_tc_tiling_on_sc=False))


_cmat = _cmat_kernel()


def _enc_body(label_r, content_r, wl_r, bl_r, wc_r, bc_r, pe_r, fe_r, w1_r,
              mcc_r, mcp_r, mcf_r, mpc_r, mfc_r):
    hx = jnp.concatenate(
        [jnp.dot(label_r[...], wl_r[...], preferred_element_type=jnp.float32)
         + bl_r[...],
         jnp.dot(content_r[...], wc_r[...], preferred_element_type=jnp.float32)
         + bc_r[...]], axis=1)
    for k, out_r in ((0, mcc_r), (1, mcp_r), (3, mcf_r)):
        out_r[...] = jnp.dot(hx, w1_r[k], preferred_element_type=jnp.float32)
    yp = jnp.dot(pe_r[...], w1_r[2], preferred_element_type=jnp.float32)
    mpc_r[...] = jnp.broadcast_to(yp, (NSM, W))
    yf = jnp.dot(fe_r[...], w1_r[4], preferred_element_type=jnp.float32)
    mfc_r[...] = jnp.broadcast_to(yf, (NSM, W))


def _norm_cc(s_val, c_val, cat):
    r = 1.0 / jnp.maximum(c_val[0] + c_val[1], 1.0)
    if cat:
        return jnp.concatenate([s_val[0] * r, s_val[1] * r], axis=1)
    return (s_val[0] + s_val[1]) * r


def _cmm(c_ref, m_ref):
    cv = c_ref[...]
    cm = cv[0] + cv[1]
    r = 1.0 / jnp.maximum(jnp.sum(cm, axis=1, keepdims=True), 1.0)
    return jnp.dot(cm, m_ref[...], preferred_element_type=jnp.float32) * r


def _make_big_body(cat_in, slab_out, has_prev, emit_x, last):
    def body(*refs):
        scc, ccc, cpc_r, cfc_r, mpc_r, mfc_r = refs[:6]
        pos = 6
        if has_prev:
            cprev = refs[pos]
            pos += 1
        w_r = refs[pos]
        pos += 1
        outs = refs[pos:]

        xn = jax.nn.relu(_norm_cc(scc[...], ccc[...], cat_in)
                         + _cmm(cpc_r, mpc_r) + _cmm(cfc_r, mfc_r))
        xc = jnp.concatenate([cprev[...], xn], axis=1) if has_prev else xn

        def store(out_r, y):
            if slab_out:
                out_r[0] = y[:, :W]
                out_r[1] = y[:, W:]
            else:
                out_r[...] = y

        store(outs[0], jnp.dot(xc, w_r[0], preferred_element_type=jnp.float32))
        if not last:
            outs[1][...] = jnp.dot(xc, w_r[1],
                                   preferred_element_type=jnp.float32)
            outs[2][...] = jnp.dot(xc, w_r[3],
                                   preferred_element_type=jnp.float32)
        if emit_x:
            outs[3][...] = xn
    return body


def _make_small_body(has_prev, emit_x):
    def body(*refs):
        ccp_r, ccf_r, mcp_r, mcf_r = refs[:4]
        pos = 4
        if has_prev:
            pprev, fprev = refs[pos:pos + 2]
            pos += 2
        w_r = refs[pos]
        pos += 1
        outs = refs[pos:]

        xpn = jax.nn.relu(_cmm(ccp_r, mcp_r))
        xfn = jax.nn.relu(_cmm(ccf_r, mcf_r))
        xp = jnp.concatenate([pprev[...], xpn], axis=1) if has_prev else xpn
        xf = jnp.concatenate([fprev[...], xfn], axis=1) if has_prev else xfn
        outs[0][...] = jnp.dot(xp, w_r[2], preferred_element_type=jnp.float32)
        outs[1][...] = jnp.dot(xf, w_r[4], preferred_element_type=jnp.float32)
        if emit_x:
            outs[2][...] = xpn
            outs[3][...] = xfn
    return body


def _dec_body(scc, ccc, cpc_r, cfc_r, mpc_r, mfc_r, w_r, b_r,
              logit_r, pred_r):
    c5 = jax.nn.relu(_norm_cc(scc[...], ccc[...], False)
                     + _cmm(cpc_r, mpc_r) + _cmm(cfc_r, mfc_r))
    lg = jnp.dot(c5, w_r[...], preferred_element_type=jnp.float32) + b_r[...]
    l0 = lg[:, 0:1]
    l1 = lg[:, 1:2]
    m = jnp.maximum(l0, l1)
    e0 = jnp.exp(l0 - m)
    e1 = jnp.exp(l1 - m)
    den = e0 + e1
    logit_r[...] = jnp.concatenate([l0, l1], axis=1)
    pred_r[...] = jnp.concatenate([e0 / den, e1 / den], axis=1)


def _cfg_spec():
    return pl.BlockSpec((2, _BLK, W), lambda i: (0, i, 0))


def _cnt_spec():
    return pl.BlockSpec((2, _BLK, 1), lambda i: (0, i, 0))


def _row_spec(width):
    return pl.BlockSpec((_BLK, width), lambda i: (i, 0))


def _full(shape):
    nd = len(shape)
    return pl.BlockSpec(shape, lambda i, _n=nd: (0,) * _n)


def _big_call(s_cc, cnt3, c_pc, c_fc, m_pc, m_fc, cprev, w,
              cat_in, slab_out, emit_x, last, win, wout):
    f32 = jnp.float32
    in_specs = [_cfg_spec(), _cnt_spec(), _cfg_spec(), _cfg_spec(),
                _full((NSM, win)), _full((NSM, win))]
    args = [s_cc, cnt3, c_pc, c_fc, m_pc, m_fc]
    has_prev = cprev is not None
    if has_prev:
        in_specs.append(_row_spec(cprev.shape[1]))
        args.append(cprev)
    in_specs.append(_full(w.shape))
    args.append(w)

    if slab_out:
        mcc_spec = pl.BlockSpec((2, _BLK, W), lambda i: (0, i, 0))
        mcc_shape = jax.ShapeDtypeStruct((2, N_CFG, W), f32)
    else:
        mcc_spec = _row_spec(wout)
        mcc_shape = jax.ShapeDtypeStruct((N_CFG, wout), f32)
    out_specs = [mcc_spec]
    out_shape = [mcc_shape]
    if not last:
        out_specs += [_row_spec(wout), _row_spec(wout)]
        out_shape += [jax.ShapeDtypeStruct((N_CFG, wout), f32)] * 2
    if emit_x:
        xw = W * (2 if cat_in else 1)
        out_specs.append(_row_spec(xw))
        out_shape.append(jax.ShapeDtypeStruct((N_CFG, xw), f32))

    return pl.pallas_call(
        _make_big_body(cat_in, slab_out, has_prev, emit_x, last),
        grid=(_GRID,),
        in_specs=in_specs,
        out_specs=out_specs,
        out_shape=out_shape,
    )(*args)


def _small_call(c_cp, c_cf, m_cp, m_cf, prevs, w, emit_x, wout, xw):
    f32 = jnp.float32
    args = [c_cp, c_cf, m_cp, m_cf]
    has_prev = prevs is not None
    if has_prev:
        args += list(prevs)
    args.append(w)
    out_shape = [jax.ShapeDtypeStruct((NSM, wout), f32)] * 2
    if emit_x:
        out_shape += [jax.ShapeDtypeStruct((NSM, xw), f32)] * 2
    return pl.pallas_call(
        _make_small_body(has_prev, emit_x),
        out_shape=out_shape,
    )(*args)


def kernel(label, content, src_cc, dst_cc, src_cp, dst_cp, src_pc, dst_pc,
           src_cf, dst_cf, src_fc, dst_fc, enc_label_W, enc_label_b,
           enc_content_W, enc_content_b, ptest_emb, ftest_emb,
           W1, W2, W3, W4, W5, dec_W, dec_b):
    f32 = jnp.float32
    scc2, dcc2 = _pad_edges(src_cc, dst_cc, EP_CC, N_CFG, 240, N_CFG)
    scp2, dcp2 = _pad_edges(src_cp, dst_cp, EP_CP, N_PT, 28, N_CFG)
    spc2, dpc2 = _pad_edges(src_pc, dst_pc, EP_PC, N_CFG, 240, NSM)
    scf2, dcf2 = _pad_edges(src_cf, dst_cf, EP_CF, N_FT, 108, N_CFG)
    sfc2, dfc2 = _pad_edges(src_fc, dst_fc, EP_FC, N_CFG, 240, NSM)
    zrows = jnp.zeros((_ZROWS, W), f32)
    zcol = jnp.zeros((_ZROWS,), f32)
    zflat = jnp.zeros((_ZFLAT,), f32)

    pc_f, fc_f, cp_f, cf_f = _cmat(zflat, spc2, dpc2, sfc2, dfc2,
                                   scp2, dcp2, scf2, dcf2)
    c_pc = pc_f.reshape(2, NPAD, 128)
    c_fc = fc_f.reshape(2, NPAD, 128)
    c_cp = cp_f.reshape(2, NSM, N_CFG)
    c_cf = cf_f.reshape(2, NSM, N_CFG)

    m1 = pl.pallas_call(
        _enc_body,
        grid=(_GRID,),
        in_specs=[pl.BlockSpec((_BLK, 128), lambda i: (i, 0)),
                  pl.BlockSpec((_BLK, 128), lambda i: (i, 0)),
                  _full((128, 64)), _full((1, 64)),
                  _full((128, 64)), _full((1, 64)),
                  _full((1, 128)), _full((1, 128)),
                  _full((5, 128, 128))],
        out_specs=[_row_spec(128), _row_spec(128), _row_spec(128),
                   _full((NSM, 128)), _full((NSM, 128))],
        out_shape=[jax.ShapeDtypeStruct((N_CFG, 128), f32)] * 3
                  + [jax.ShapeDtypeStruct((NSM, 128), f32)] * 2,
    )(label, content, enc_label_W, enc_label_b.reshape(1, 64),
      enc_content_W, enc_content_b.reshape(1, 64),
      ptest_emb.reshape(1, 128), ftest_emb.reshape(1, 128), W1)
    m1_cc, m1_cp, m1_cf, m1_pc, m1_fc = m1

    s1, cnt_cc = _agg_es_counts(zrows, zcol, m1_cc, scc2, dcc2)
    cnt3 = cnt_cc.reshape(2, NPAD, 1)

    m2 = _big_call(s1, cnt3, c_pc, c_fc, m1_pc, m1_fc, None, W2,
                   False, False, True, False, 128, 128)
    m2_cc, m2_cp, m2_cf, c1 = m2
    sm2 = _small_call(c_cp, c_cf, m1_cp, m1_cf, None, W2, True, 128, 128)
    m2_pc, m2_fc, p1, f1 = sm2
    s2 = _agg_es(zrows, zcol, m2_cc, scc2, dcc2)

    m3 = _big_call(s2, cnt3, c_pc, c_fc, m2_pc, m2_fc, c1, W3,
                   False, True, False, False, 128, 256)
    m3_cc, m3_cp, m3_cf = m3
    sm3 = _small_call(c_cp, c_cf, m2_cp, m2_cf, (p1, f1), W3, False, 256, 0)
    m3_pc, m3_fc = sm3
    s3 = _agg_cs(zrows, zcol, m3_cc, scc2, dcc2)

    m4 = _big_call(s3, cnt3, c_pc, c_fc, m3_pc, m3_fc, None, W4,
                   True, True, True, False, 256, 256)
    m4_cc, m4_cp, m4_cf, c3 = m4
    sm4 = _small_call(c_cp, c_cf, m3_cp, m3_cf, None, W4, True, 256, 256)
    m4_pc, m4_fc, p3, f3 = sm4
    s4 = _agg_cs(zrows, zcol, m4_cc, scc2, dcc2)

    m5 = _big_call(s4, cnt3, c_pc, c_fc, m4_pc, m4_fc, c3, W5,
                   True, False, False, True, 256, 128)
    m5_cc = m5[0]
    sm5 = _small_call(c_cp, c_cf, m4_cp, m4_cf, (p3, f3), W5, False, 128, 0)
    m5_pc, m5_fc = sm5
    s5 = _agg_es(zrows, zcol, m5_cc, scc2, dcc2)

    w_pad = jnp.zeros((128, 128), f32).at[:, :2].set(dec_W)
    b_pad = jnp.zeros((1, 128), f32).at[0, :2].set(dec_b)
    logits, pred = pl.pallas_call(
        _dec_body,
        grid=(_GRID,),
        in_specs=[_cfg_spec(), _cnt_spec(), _cfg_spec(), _cfg_spec(),
                  _full((NSM, 128)), _full((NSM, 128)),
                  _full((128, 128)), _full((1, 128))],
        out_specs=[pl.BlockSpec((_BLK, 2), lambda i: (i, 0)),
                   pl.BlockSpec((_BLK, 2), lambda i: (i, 0))],
        out_shape=[jax.ShapeDtypeStruct((N_CFG, 2), f32),
                   jax.ShapeDtypeStruct((N_CFG, 2), f32)],
    )(s5, cnt3, c_pc, c_fc, m5_pc, m5_fc, w_pad, b_pad)
    return logits, pred
